# Initial kernel scaffold; baseline (speedup 1.0000x reference)
import jax
import jax.numpy as jnp
from jax.experimental import pallas as pl


def kernel(logits):
    raise NotImplementedError("write your pallas kernel here")


# speedup vs baseline: 22.5313x; 22.5313x over previous
import jax
import jax.numpy as jnp
from jax.experimental import pallas as pl

_B = 64
_V = 100000
_K = 10000
_R = 8


def _body(x_ref, g_ref, probs_ref, samp_ref):
    min32 = jnp.int32(-2147483648)
    x = x_ref[...]
    b = jax.lax.bitcast_convert_type(x, jnp.int32)
    s = jnp.where(b < 0, ~b ^ min32, b)

    def step(i, p_u):
        bit = jnp.left_shift(jnp.int32(1), jnp.int32(31) - i)
        t_u = p_u | bit
        t_s = t_u ^ min32
        cnt = jnp.sum((s >= t_s).astype(jnp.int32), axis=1, keepdims=True)
        return jnp.where(cnt >= _K, t_u, p_u)

    p_u = jax.lax.fori_loop(0, 32, step, jnp.zeros((_R, 1), jnp.int32))
    t_s = p_u ^ min32
    mask = s >= t_s

    m = jnp.max(x, axis=1, keepdims=True)
    e = jnp.where(mask, jnp.exp(x - m), 0.0)
    denom = jnp.sum(e, axis=1, keepdims=True)
    probs = e / denom
    probs_ref[...] = probs

    v = jnp.log(probs + 1e-20) + g_ref[...]
    vm = jnp.max(v, axis=1, keepdims=True)
    iota = jax.lax.broadcasted_iota(jnp.int32, v.shape, 1)
    idx = jnp.min(jnp.where(v == vm, iota, jnp.int32(2**31 - 1)), axis=1,
                  keepdims=True)
    samp_ref[...] = idx


def kernel(logits):
    gumbel = jax.random.gumbel(jax.random.key(42), (_B, _V), jnp.float32)
    probs, samples = pl.pallas_call(
        _body,
        grid=(_B // _R,),
        in_specs=[
            pl.BlockSpec((_R, _V), lambda i: (i, 0)),
            pl.BlockSpec((_R, _V), lambda i: (i, 0)),
        ],
        out_specs=[
            pl.BlockSpec((_R, _V), lambda i: (i, 0)),
            pl.BlockSpec((_R, 1), lambda i: (i, 0)),
        ],
        out_shape=[
            jax.ShapeDtypeStruct((_B, _V), jnp.float32),
            jax.ShapeDtypeStruct((_B, 1), jnp.int32),
        ],
    )(logits, gumbel)
    return samples, probs
